# baseline (speedup 1.0000x reference)
import functools

import jax
import jax.numpy as jnp
from jax import lax
from jax.experimental import pallas as pl
from jax.experimental.pallas import tpu as pltpu
from jax.experimental.pallas import tpu_sc as plsc

B, L, D = 128, 8192, 4
LD = L * D
NC, NS = 2, 16
NW = NC * NS
BPW = B // NW
NGRP = LD // 16

_mesh = plsc.VectorSubcoreMesh(core_axis_name="c", subcore_axis_name="s")


NCH = 1
CH = LD // NCH
CHQ = L // NCH


@functools.partial(
    pl.kernel,
    mesh=_mesh,
    compiler_params=pltpu.CompilerParams(needs_layout_passes=False),
    out_type=jax.ShapeDtypeStruct((B * LD,), jnp.float32),
    scratch_types=[
        pltpu.VMEM((L,), jnp.int32),
        pltpu.VMEM((LD,), jnp.float32),
        pltpu.VMEM((CH,), jnp.float32),
        pltpu.VMEM((CH,), jnp.float32),
        pltpu.SemaphoreType.DMA,
        pltpu.SemaphoreType.DMA,
        pltpu.SemaphoreType.DMA,
        pltpu.SemaphoreType.DMA,
    ],
)
def _interleave_sc(in_hbm, qr_hbm, out_hbm, qr_v, x0_v,
                   y0_v, y1_v,
                   fp_sem, in_sem0,
                   out_sem0, out_sem1):
    wid = lax.axis_index("s") * NC + lax.axis_index("c")
    xbufs = (x0_v,)
    ybufs = (y0_v, y1_v)
    in_sems = (in_sem0,)
    out_sems = (out_sem0, out_sem1)
    base = wid * BPW

    fp_cp = pltpu.async_copy(qr_hbm, qr_v, fp_sem)
    in_cp = [None]
    in_cp[0] = pltpu.async_copy(
        in_hbm.at[pl.ds(base * LD, LD)], xbufs[0], in_sems[0]
    )
    fp_cp.wait()

    out_cp = [None, None]
    for bi in range(BPW):
        in_cp[0].wait()
        xb = xbufs[0]
        for c in range(NCH):
            cc = bi % 2
            yb = ybufs[cc]
            if out_cp[cc] is not None:
                out_cp[cc].wait()

            @plsc.parallel_loop(0, CHQ, step=16, unroll=8)
            def permute_grp(o):
                idx = qr_v[pl.ds(c * CHQ + o, 16)]
                ybase = (o // 128) * (D * 128) + (o % 128)
                for d in range(D):
                    yb[pl.ds(ybase + d * 128, 16)] = plsc.load_gather(
                        xb, [idx + d * 128] if d else [idx]
                    )

            out_cp[cc] = pltpu.async_copy(
                yb,
                out_hbm.at[pl.ds((base + bi) * LD + c * CH, CH)],
                out_sems[cc],
            )
        if bi + 1 < BPW:
            in_cp[0] = pltpu.async_copy(
                in_hbm.at[pl.ds((base + bi + 1) * LD, LD)],
                xbufs[0],
                in_sems[0],
            )
    for cp in out_cp:
        if cp is not None:
            cp.wait()


def kernel(inputs, p_array):
    z = inputs.reshape(B, L // 128, 128, D).transpose(0, 1, 3, 2).reshape(B * LD)
    p2 = p_array.reshape(L // 128, 128)
    qr = ((p2 // 128) * (D * 128) + (p2 % 128)).reshape(L)
    out = _interleave_sc(z, qr)
    return (
        out.reshape(B, L // 128, D, 128).transpose(0, 1, 3, 2).reshape(B, L, D)
    )


# speedup vs baseline: 1.1697x; 1.1697x over previous
import functools

import jax
import jax.numpy as jnp
from jax import lax
from jax.experimental import pallas as pl
from jax.experimental.pallas import tpu as pltpu
from jax.experimental.pallas import tpu_sc as plsc

B, L, D = 128, 8192, 4
LD = L * D
NC, NS = 2, 16
NW = NC * NS
BPW = B // NW
NGRP = LD // 16

_mesh = plsc.VectorSubcoreMesh(core_axis_name="c", subcore_axis_name="s")


NCH = 2
CH = LD // NCH
CHQ = L // NCH


@functools.partial(
    pl.kernel,
    mesh=_mesh,
    compiler_params=pltpu.CompilerParams(needs_layout_passes=False),
    out_type=jax.ShapeDtypeStruct((B * LD,), jnp.float32),
    scratch_types=[
        pltpu.VMEM((L,), jnp.int32),
        pltpu.VMEM((LD,), jnp.float32),
        pltpu.VMEM((LD,), jnp.float32),
        pltpu.VMEM((CH,), jnp.float32),
        pltpu.VMEM((CH,), jnp.float32),
        pltpu.SemaphoreType.DMA,
        pltpu.SemaphoreType.DMA,
        pltpu.SemaphoreType.DMA,
        pltpu.SemaphoreType.DMA,
        pltpu.SemaphoreType.DMA,
    ],
)
def _interleave_sc(in_hbm, qr_hbm, out_hbm, qr_v, x0_v, x1_v,
                   y0_v, y1_v,
                   fp_sem, in_sem0, in_sem1,
                   out_sem0, out_sem1):
    wid = lax.axis_index("s") * NC + lax.axis_index("c")
    xbufs = (x0_v, x1_v)
    ybufs = (y0_v, y1_v)
    in_sems = (in_sem0, in_sem1)
    out_sems = (out_sem0, out_sem1)
    base = wid * BPW

    fp_cp = pltpu.async_copy(qr_hbm, qr_v, fp_sem)
    in_cp = [None, None]
    in_cp[0] = pltpu.async_copy(
        in_hbm.at[pl.ds(base * LD, LD)], xbufs[0], in_sems[0]
    )
    fp_cp.wait()

    out_cp = [None, None]
    for bi in range(BPW):
        cur = bi % 2
        in_cp[cur].wait()
        if bi + 1 < BPW:
            nxt = (bi + 1) % 2
            in_cp[nxt] = pltpu.async_copy(
                in_hbm.at[pl.ds((base + bi + 1) * LD, LD)],
                xbufs[nxt],
                in_sems[nxt],
            )
        xb = xbufs[cur]
        for c in range(NCH):
            cc = c % 2
            yb = ybufs[cc]
            if out_cp[cc] is not None:
                out_cp[cc].wait()

            @plsc.parallel_loop(0, CHQ, step=16, unroll=8)
            def permute_grp(o):
                idx = qr_v[pl.ds(c * CHQ + o, 16)]
                ybase = (o // 128) * (D * 128) + (o % 128)
                for d in range(D):
                    yb[pl.ds(ybase + d * 128, 16)] = plsc.load_gather(
                        xb, [idx + d * 128] if d else [idx]
                    )

            out_cp[cc] = pltpu.async_copy(
                yb,
                out_hbm.at[pl.ds((base + bi) * LD + c * CH, CH)],
                out_sems[cc],
            )
    for cp in out_cp:
        if cp is not None:
            cp.wait()


def kernel(inputs, p_array):
    z = inputs.reshape(B, L // 128, 128, D).transpose(0, 1, 3, 2).reshape(B * LD)
    p2 = p_array.reshape(L // 128, 128)
    qr = ((p2 // 128) * (D * 128) + (p2 % 128)).reshape(L)
    out = _interleave_sc(z, qr)
    return (
        out.reshape(B, L // 128, D, 128).transpose(0, 1, 3, 2).reshape(B, L, D)
    )
